# baseline (speedup 1.0000x reference)
import jax
import jax.numpy as jnp
from jax.experimental import pallas as pl
from jax.experimental.pallas import tpu as pltpu

H, W, C = 64, 64, 21
N = H * W
THETA_ALPHA, THETA_BETA, THETA_GAMMA = 80.0, 0.05, 3.0
BIL_COMPAT, SP_COMPAT = 10.0, 3.0
NUM_ITERS = 5

RB = 256
NBLK = N // RB
NB_DIMS = 5
NS_DIMS = 2
ND = NB_DIMS + NS_DIMS


def _softmax0(x):
    m = jnp.max(x, axis=0, keepdims=True)
    e = jnp.exp(x - m)
    s = jnp.sum(e, axis=0, keepdims=True)
    return e / s


def _crf_body(f7c_ref, f7r_ref, ut_ref, out_ref, m_scr, mm_scr):
    fr_b = f7r_ref[0:NB_DIMS, :]
    fr_s = f7r_ref[NB_DIMS:ND, :]
    hr_b = 0.5 * jnp.sum(fr_b * fr_b, axis=0, keepdims=True)
    hr_s = 0.5 * jnp.sum(fr_s * fr_s, axis=0, keepdims=True)
    fr16f = [f7r_ref[k:k + 1, :].astype(jnp.bfloat16).astype(jnp.float32)
             for k in range(ND)]

    def _exp_tiles(fblk):
        fb = fblk[:, 0:NB_DIMS]
        fs = fblk[:, NB_DIMS:ND]
        hc_b = 0.5 * jnp.sum(fb * fb, axis=1, keepdims=True)
        hc_s = 0.5 * jnp.sum(fs * fs, axis=1, keepdims=True)
        fc16f = fblk.astype(jnp.bfloat16).astype(jnp.float32)
        acc_b = -(hc_b + hr_b)
        for k in range(NB_DIMS):
            acc_b = acc_b + fc16f[:, k:k + 1] * fr16f[k]
        acc_s = -(hc_s + hr_s)
        for k in range(NB_DIMS, ND):
            acc_s = acc_s + fc16f[:, k:k + 1] * fr16f[k]
        return (jnp.exp(jnp.minimum(acc_b, 0.0)),
                jnp.exp(jnp.minimum(acc_s, 0.0)))

    def phase_a(b, carry):
        cs_b, cs_s = carry
        r0 = pl.multiple_of(b * RB, RB)
        eb, es = _exp_tiles(f7c_ref[pl.ds(r0, RB), :])
        m_scr[pl.ds(r0, RB), :] = eb.astype(jnp.bfloat16)
        return (cs_b + jnp.sum(eb, axis=0, keepdims=True),
                cs_s + jnp.sum(es, axis=0, keepdims=True))

    cs_b, cs_s = jax.lax.fori_loop(
        0, NBLK, phase_a,
        (jnp.zeros((1, N), jnp.float32), jnp.zeros((1, N), jnp.float32)))

    sqb, sqs = jnp.sqrt(BIL_COMPAT), jnp.sqrt(SP_COMPAT)
    cbr = sqb / (jnp.sqrt(cs_b) + 1e-20)
    csr = sqs / (jnp.sqrt(cs_s) + 1e-20)

    def phase_b(b, carry):
        r0 = pl.multiple_of(b * RB, RB)
        _, es = _exp_tiles(f7c_ref[pl.ds(r0, RB), :])
        eb = m_scr[pl.ds(r0, RB), :].astype(jnp.float32)
        cbc = sqb / (jnp.sqrt(jnp.sum(eb, axis=1, keepdims=True)) + 1e-20)
        csc = sqs / (jnp.sqrt(jnp.sum(es, axis=1, keepdims=True)) + 1e-20)
        m = (cbc * cbr) * eb + (csc * csr) * es
        m_scr[pl.ds(r0, RB), :] = m.astype(jnp.bfloat16)
        return carry

    jax.lax.fori_loop(0, NBLK, phase_b, 0)

    ut = ut_ref[...]
    qt = _softmax0(-ut)
    KB = 512
    for _ in range(NUM_ITERS):
        qb = qt.astype(jnp.bfloat16)

        def mm_step(b, carry, qb=qb):
            c0 = pl.multiple_of(b * (2 * KB), 2 * KB)
            mm_scr[:, pl.ds(c0, KB)] = jnp.dot(
                qb, m_scr[:, pl.ds(c0, KB)],
                preferred_element_type=jnp.float32)
            c1 = pl.multiple_of(c0 + KB, KB)
            mm_scr[:, pl.ds(c1, KB)] = jnp.dot(
                qb, m_scr[:, pl.ds(c1, KB)],
                preferred_element_type=jnp.float32)
            return carry

        jax.lax.fori_loop(0, N // (2 * KB), mm_step, 0)
        qt = _softmax0(mm_scr[...] - ut)
    out_ref[...] = qt


def _build_features(image):
    ys, xs = jnp.meshgrid(jnp.arange(H, dtype=jnp.float32),
                          jnp.arange(W, dtype=jnp.float32), indexing='ij')
    xs = xs.reshape(N, 1)
    ys = ys.reshape(N, 1)
    color = image.reshape(N, 3) / THETA_BETA
    return jnp.concatenate(
        [xs / THETA_ALPHA, ys / THETA_ALPHA, color,
         xs / THETA_GAMMA, ys / THETA_GAMMA], axis=1)


def kernel(unary, image):
    f7 = _build_features(image)
    ut = unary.reshape(N, C).T
    out_t = pl.pallas_call(
        _crf_body,
        out_shape=jax.ShapeDtypeStruct((C, N), jnp.float32),
        scratch_shapes=[pltpu.VMEM((N, N), jnp.bfloat16),
                        pltpu.VMEM((C, N), jnp.float32)],
        compiler_params=pltpu.CompilerParams(
            vmem_limit_bytes=64 * 1024 * 1024),
    )(f7, f7.T, ut)
    return out_t.T.reshape(H, W, C)


# speedup vs baseline: 1.5733x; 1.5733x over previous
import jax
import jax.numpy as jnp
from jax.experimental import pallas as pl
from jax.experimental.pallas import tpu as pltpu

H, W, C = 64, 64, 21
N = H * W
THETA_ALPHA, THETA_BETA, THETA_GAMMA = 80.0, 0.05, 3.0
BIL_COMPAT, SP_COMPAT = 10.0, 3.0
NUM_ITERS = 5

RB = 512
NBLK = N // RB
NB_DIMS = 5
NS_DIMS = 2
ND = NB_DIMS + NS_DIMS


def _softmax0(x):
    m = jnp.max(x, axis=0, keepdims=True)
    e = jnp.exp(x - m)
    s = jnp.sum(e, axis=0, keepdims=True)
    return e / s


def _crf_body(f7c_ref, f7r_ref, ut_ref, out_ref, m_scr, mm_scr):
    fr_b = f7r_ref[0:NB_DIMS, :]
    fr_s = f7r_ref[NB_DIMS:ND, :]
    hr_b = 0.5 * jnp.sum(fr_b * fr_b, axis=0, keepdims=True)
    hr_s = 0.5 * jnp.sum(fr_s * fr_s, axis=0, keepdims=True)
    frb16 = fr_b.astype(jnp.bfloat16)
    frs16 = fr_s.astype(jnp.bfloat16)

    def _exp_tiles(fblk):
        fb = fblk[:, 0:NB_DIMS]
        fs = fblk[:, NB_DIMS:ND]
        hc_b = 0.5 * jnp.sum(fb * fb, axis=1, keepdims=True)
        hc_s = 0.5 * jnp.sum(fs * fs, axis=1, keepdims=True)
        dot_b = jnp.dot(fb.astype(jnp.bfloat16), frb16,
                        preferred_element_type=jnp.float32)
        dot_s = jnp.dot(fs.astype(jnp.bfloat16), frs16,
                        preferred_element_type=jnp.float32)
        ab = jnp.minimum((dot_b - hc_b) - hr_b, 0.0)
        as_ = jnp.minimum((dot_s - hc_s) - hr_s, 0.0)
        return jnp.exp(ab), jnp.exp(as_)

    def phase_a(b, carry):
        cs_b, cs_s = carry
        r0 = pl.multiple_of(b * RB, RB)
        eb, es = _exp_tiles(f7c_ref[pl.ds(r0, RB), :])
        m_scr[pl.ds(r0, RB), :] = eb.astype(jnp.bfloat16)
        return (cs_b + jnp.sum(eb, axis=0, keepdims=True),
                cs_s + jnp.sum(es, axis=0, keepdims=True))

    cs_b, cs_s = jax.lax.fori_loop(
        0, NBLK, phase_a,
        (jnp.zeros((1, N), jnp.float32), jnp.zeros((1, N), jnp.float32)))

    sqb, sqs = jnp.sqrt(BIL_COMPAT), jnp.sqrt(SP_COMPAT)
    cbr = sqb / (jnp.sqrt(cs_b) + 1e-20)
    csr = sqs / (jnp.sqrt(cs_s) + 1e-20)

    def phase_b(b, carry):
        r0 = pl.multiple_of(b * RB, RB)
        _, es = _exp_tiles(f7c_ref[pl.ds(r0, RB), :])
        eb = m_scr[pl.ds(r0, RB), :].astype(jnp.float32)
        cbc = sqb / (jnp.sqrt(jnp.sum(eb, axis=1, keepdims=True)) + 1e-20)
        csc = sqs / (jnp.sqrt(jnp.sum(es, axis=1, keepdims=True)) + 1e-20)
        m = (cbc * cbr) * eb + (csc * csr) * es
        m_scr[pl.ds(r0, RB), :] = m.astype(jnp.bfloat16)
        return carry

    jax.lax.fori_loop(0, NBLK, phase_b, 0)

    ut = ut_ref[...]
    qt = _softmax0(-ut)
    KB = 512
    for _ in range(NUM_ITERS):
        qb = qt.astype(jnp.bfloat16)

        def mm_step(b, carry, qb=qb):
            c0 = pl.multiple_of(b * (2 * KB), 2 * KB)
            mm_scr[:, pl.ds(c0, KB)] = jnp.dot(
                qb, m_scr[:, pl.ds(c0, KB)],
                preferred_element_type=jnp.float32)
            c1 = pl.multiple_of(c0 + KB, KB)
            mm_scr[:, pl.ds(c1, KB)] = jnp.dot(
                qb, m_scr[:, pl.ds(c1, KB)],
                preferred_element_type=jnp.float32)
            return carry

        jax.lax.fori_loop(0, N // (2 * KB), mm_step, 0)
        qt = _softmax0(mm_scr[...] - ut)
    out_ref[...] = qt


def _build_features(image):
    ys, xs = jnp.meshgrid(jnp.arange(H, dtype=jnp.float32),
                          jnp.arange(W, dtype=jnp.float32), indexing='ij')
    xs = xs.reshape(N, 1)
    ys = ys.reshape(N, 1)
    color = image.reshape(N, 3) / THETA_BETA
    return jnp.concatenate(
        [xs / THETA_ALPHA, ys / THETA_ALPHA, color,
         xs / THETA_GAMMA, ys / THETA_GAMMA], axis=1)


def kernel(unary, image):
    f7 = _build_features(image)
    ut = unary.reshape(N, C).T
    out_t = pl.pallas_call(
        _crf_body,
        out_shape=jax.ShapeDtypeStruct((C, N), jnp.float32),
        scratch_shapes=[pltpu.VMEM((N, N), jnp.bfloat16),
                        pltpu.VMEM((C, N), jnp.float32)],
        compiler_params=pltpu.CompilerParams(
            vmem_limit_bytes=64 * 1024 * 1024),
    )(f7, f7.T, ut)
    return out_t.T.reshape(H, W, C)
